# baseline (speedup 1.0000x reference)
import functools

import jax
import jax.numpy as jnp
from jax import lax
from jax.experimental import pallas as pl
from jax.experimental.pallas import tpu as pltpu
from jax.experimental.pallas import tpu_sc as plsc

_F = 64
_NSPECIES = 14
_DIMS = (1, 3, 5)
_L = 128


def _pair_table(leq, k):
    w = _F * k
    t = leq.reshape(_NSPECIES, w)
    ta = jnp.broadcast_to(t[:, None, :], (_NSPECIES, _NSPECIES, w))
    tb = jnp.broadcast_to(t[None, :, :], (_NSPECIES, _NSPECIES, w))
    tp = jnp.concatenate([ta, tb], axis=-1).reshape(-1, _L)
    pad = (-tp.shape[0]) % 8
    return jnp.pad(tp, ((0, pad), (0, 0)))


def _gather3(idx0, idx1, idx2, t0, t1, t2):
    info = plsc.get_sparse_core_info()
    nc, ns = info.num_cores, info.num_subcores
    nw = nc * ns
    nch = tuple(i.size // nw // _L for i in (idx0, idx1, idx2))

    @functools.partial(
        pl.kernel,
        mesh=plsc.VectorSubcoreMesh(core_axis_name="c", subcore_axis_name="s"),
        compiler_params=pltpu.CompilerParams(use_tc_tiling_on_sc=False),
        out_type=[
            jax.ShapeDtypeStruct((idx0.size, _L), jnp.float32),
            jax.ShapeDtypeStruct((idx1.size, _L), jnp.float32),
            jax.ShapeDtypeStruct((idx2.size, _L), jnp.float32),
        ],
        scratch_types=[
            pltpu.VMEM((nch[0], _L), jnp.int32),
            pltpu.VMEM((nch[1], _L), jnp.int32),
            pltpu.VMEM((nch[2], _L), jnp.int32),
            pltpu.VMEM((2, _L, _L), jnp.float32),
            pltpu.SemaphoreType.DMA,
            pltpu.SemaphoreType.DMA,
        ],
    )
    def k(i0_hbm, i1_hbm, i2_hbm, t0_hbm, t1_hbm, t2_hbm,
          o0_hbm, o1_hbm, o2_hbm, i0_v, i1_v, i2_v, rbuf, sem0, sem1):
        wid = lax.axis_index("s") * nc + lax.axis_index("c")
        pltpu.sync_copy(i0_hbm.at[pl.ds(wid * nch[0], nch[0])], i0_v)
        pltpu.sync_copy(i1_hbm.at[pl.ds(wid * nch[1], nch[1])], i1_v)
        pltpu.sync_copy(i2_hbm.at[pl.ds(wid * nch[2], nch[2])], i2_v)
        sems = (sem0, sem1)

        def phase(tbl, idx_v, n, out):
            base = wid * n * _L

            def start(ci, b):
                pltpu.async_copy(tbl.at[idx_v.at[ci]], rbuf.at[b], sems[b])

            start(0, 0)

            def body(g, carry):
                for b in (0, 1):
                    ci = 2 * g + b

                    @pl.when(ci + 1 < n)
                    def _():
                        start(ci + 1, 1 - b)

                    pltpu.make_async_copy(
                        tbl.at[idx_v.at[ci]], rbuf.at[b], sems[b]).wait()
                    pltpu.sync_copy(
                        rbuf.at[b], out.at[pl.ds(base + ci * _L, _L)])
                return carry

            lax.fori_loop(0, n // 2, body, 0)

        phase(t0_hbm, i0_v, nch[0], o0_hbm)
        phase(t1_hbm, i1_v, nch[1], o1_hbm)
        phase(t2_hbm, i2_v, nch[2], o2_hbm)

    return k(idx0, idx1, idx2, t0, t1, t2)


def kernel(Z, leq0, leq1, leq2):
    N = Z.shape[0]
    zpair = Z.astype(jnp.int32).reshape(-1, 2)
    zidx = zpair[:, 0] * _NSPECIES + zpair[:, 1]
    r1 = jnp.arange(3, dtype=jnp.int32)
    r2 = jnp.arange(5, dtype=jnp.int32)
    idx0 = zidx.reshape(-1, _L)
    idx1 = (zidx[:, None] * 3 + r1).reshape(-1, _L)
    idx2 = (zidx[:, None] * 5 + r2).reshape(-1, _L)
    t0 = _pair_table(leq0, _DIMS[0])
    t1 = _pair_table(leq1, _DIMS[1])
    t2 = _pair_table(leq2, _DIMS[2])
    o0, o1, o2 = _gather3(idx0, idx1, idx2, t0, t1, t2)
    return (
        o0.reshape(N, _F, 1),
        o1.reshape(N, _F, 3),
        o2.reshape(N, _F, 5),
        jnp.zeros((N, _F, 7), jnp.float32),
        jnp.zeros((N, _F, 9), jnp.float32),
        jnp.zeros((N, _F, 11), jnp.float32),
    )


# speedup vs baseline: 1.0212x; 1.0212x over previous
import functools

import jax
import jax.numpy as jnp
from jax import lax
from jax.experimental import pallas as pl
from jax.experimental.pallas import tpu as pltpu
from jax.experimental.pallas import tpu_sc as plsc

_F = 64
_NSPECIES = 14
_DIMS = (1, 3, 5)
_C = 32
_LI = 128


def _pair_table(leq, k):
    w = _F * k
    t = leq.reshape(_NSPECIES, w)
    ta = jnp.broadcast_to(t[:, None, :], (_NSPECIES, _NSPECIES, w))
    tb = jnp.broadcast_to(t[None, :, :], (_NSPECIES, _NSPECIES, w))
    tp = jnp.concatenate([ta, tb], axis=-1).reshape(_NSPECIES * _NSPECIES, 2 * w)
    pad = (-tp.shape[0]) % 8
    return jnp.pad(tp, ((0, pad), (0, 0)))


def _gather3(zp, t0, t1, t2):
    npair = zp.size
    info = plsc.get_sparse_core_info()
    nc, ns = info.num_cores, info.num_subcores
    nw = nc * ns
    bw = npair // nw
    nch = bw // _C
    irows = bw // _LI
    cpr = _LI // _C
    d0, d1, d2 = (2 * _F * k for k in _DIMS)

    @functools.partial(
        pl.kernel,
        mesh=plsc.VectorSubcoreMesh(core_axis_name="c", subcore_axis_name="s"),
        compiler_params=pltpu.CompilerParams(use_tc_tiling_on_sc=False),
        out_type=[
            jax.ShapeDtypeStruct((npair, d0), jnp.float32),
            jax.ShapeDtypeStruct((npair, d1), jnp.float32),
            jax.ShapeDtypeStruct((npair, d2), jnp.float32),
        ],
        scratch_types=[
            pltpu.VMEM((irows, _LI), jnp.int32),
            pltpu.VMEM((2, _C, d0), jnp.float32),
            pltpu.VMEM((2, _C, d1), jnp.float32),
            pltpu.VMEM((2, _C, d2), jnp.float32),
            pltpu.SemaphoreType.DMA,
            pltpu.SemaphoreType.DMA,
        ],
    )
    def k(zp_hbm, t0_hbm, t1_hbm, t2_hbm, o0_hbm, o1_hbm, o2_hbm,
          idx_v, r0, r1, r2, sem0, sem1):
        wid = lax.axis_index("s") * nc + lax.axis_index("c")
        base = wid * bw
        pltpu.sync_copy(zp_hbm.at[pl.ds(wid * irows, irows)], idx_v)
        sems = (sem0, sem1)

        def chunk_idx(ci):
            return idx_v.at[ci // cpr, pl.ds((ci % cpr) * _C, _C)]

        def start(ci, b):
            idx = chunk_idx(ci)
            pltpu.async_copy(t0_hbm.at[idx], r0.at[b], sems[b])
            pltpu.async_copy(t1_hbm.at[idx], r1.at[b], sems[b])
            pltpu.async_copy(t2_hbm.at[idx], r2.at[b], sems[b])

        start(0, 0)

        def body(g, carry):
            for b in (0, 1):
                ci = 2 * g + b

                @pl.when(ci + 1 < nch)
                def _():
                    start(ci + 1, 1 - b)

                idx = chunk_idx(ci)
                pltpu.make_async_copy(t0_hbm.at[idx], r0.at[b], sems[b]).wait()
                pltpu.make_async_copy(t1_hbm.at[idx], r1.at[b], sems[b]).wait()
                pltpu.make_async_copy(t2_hbm.at[idx], r2.at[b], sems[b]).wait()
                off = base + ci * _C
                pltpu.sync_copy(r0.at[b], o0_hbm.at[pl.ds(off, _C)])
                pltpu.sync_copy(r1.at[b], o1_hbm.at[pl.ds(off, _C)])
                pltpu.sync_copy(r2.at[b], o2_hbm.at[pl.ds(off, _C)])
            return carry

        lax.fori_loop(0, nch // 2, body, 0)

    return k(zp, t0, t1, t2)


def kernel(Z, leq0, leq1, leq2):
    N = Z.shape[0]
    zpair = Z.astype(jnp.int32).reshape(-1, 2)
    zidx = zpair[:, 0] * _NSPECIES + zpair[:, 1]
    zp = zidx.reshape(-1, _LI)
    t0 = _pair_table(leq0, _DIMS[0])
    t1 = _pair_table(leq1, _DIMS[1])
    t2 = _pair_table(leq2, _DIMS[2])
    o0, o1, o2 = _gather3(zp, t0, t1, t2)
    return (
        o0.reshape(N, _F, 1),
        o1.reshape(N, _F, 3),
        o2.reshape(N, _F, 5),
        jnp.zeros((N, _F, 7), jnp.float32),
        jnp.zeros((N, _F, 9), jnp.float32),
        jnp.zeros((N, _F, 11), jnp.float32),
    )


# speedup vs baseline: 4.8736x; 4.7724x over previous
import functools

import jax
import jax.numpy as jnp
from jax import lax
from jax.experimental import pallas as pl
from jax.experimental.pallas import tpu as pltpu
from jax.experimental.pallas import tpu_sc as plsc

_F = 64
_NSPECIES = 14
_DIMS = (1, 3, 5)
_W = 640
_NROW = 16


def _lut_pack(leq0, leq1, leq2):
    lut = jnp.concatenate(
        [leq.reshape(_NSPECIES, _F * k) for leq, k in
         zip((leq0, leq1, leq2), _DIMS)], axis=1)
    lut = jnp.pad(lut, ((0, _NROW - _NSPECIES), (0, _W - lut.shape[1])))
    return lut.reshape(-1)


def _gather3(z, lut):
    n = z.shape[0]
    info = plsc.get_sparse_core_info()
    nc, ns = info.num_cores, info.num_subcores
    nw = nc * ns
    bw = n // nw
    nbl = bw // 128
    nblocks = n // 128

    @functools.partial(
        pl.kernel,
        mesh=plsc.VectorSubcoreMesh(core_axis_name="c", subcore_axis_name="s"),
        compiler_params=pltpu.CompilerParams(
            use_tc_tiling_on_sc=False, needs_layout_passes=False),
        out_type=[
            jax.ShapeDtypeStruct((_F, 1, n), jnp.float32),
            jax.ShapeDtypeStruct((3, _F // 8, nblocks, 8, 128), jnp.float32),
            jax.ShapeDtypeStruct((5, _F // 8, nblocks, 8, 128), jnp.float32),
        ],
        scratch_types=[
            pltpu.VMEM((bw,), jnp.int32),
            pltpu.VMEM((_NROW * _W,), jnp.float32),
            pltpu.VMEM((2, 8, 8, 128), jnp.float32),
            pltpu.VMEM((2, 8, 1, 1024), jnp.float32),
            pltpu.SemaphoreType.DMA,
            pltpu.SemaphoreType.DMA,
        ],
    )
    def k(z_hbm, lut_hbm, o0_hbm, o1_hbm, o2_hbm, zv, lutv, stg, stg0,
          sem0, sem1):
        wid = lax.axis_index("s") * nc + lax.axis_index("c")
        nb0 = wid * nbl
        pltpu.sync_copy(z_hbm.at[pl.ds(wid * bw, bw)], zv)
        pltpu.sync_copy(lut_hbm, lutv)
        sems = (sem0, sem1)

        def compute(buf, col0, cstride, f_major):
            def blk(nb, carry):
                for l in range(8):
                    zvec = zv[pl.ds(nb * 128 + l * 16, 16)]
                    zbase = zvec * _W
                    for fi in range(8):
                        idx = zbase + (col0 + fi * cstride)
                        v = plsc.load_gather(lutv, [idx])
                        if f_major:
                            stg0[buf, fi, 0, pl.ds(nb * 128 + l * 16, 16)] = v
                        else:
                            stg[buf, nb, fi, pl.ds(l * 16, 16)] = v
                return carry

            lax.fori_loop(0, nbl, blk, 0)

        def section(nloop, out_dst, col_of, cstride, f_major):
            buf = stg0 if f_major else stg

            def body2(g, carry):
                for par in range(2):
                    i = 2 * g + par

                    @pl.when(i >= 2)
                    def _():
                        pltpu.make_async_copy(
                            buf.at[par], out_dst(i - 2), sems[par]).wait()

                    compute(par, col_of(i), cstride, f_major)
                    pltpu.async_copy(buf.at[par], out_dst(i), sems[par])
                return carry

            lax.fori_loop(0, nloop // 2, body2, 0)
            for par in range(2):
                i = nloop - 2 + par
                pltpu.make_async_copy(buf.at[par], out_dst(i), sems[par]).wait()

        section(
            8,
            lambda i: o0_hbm.at[pl.ds(i * 8, 8), pl.ds(0, 1),
                                pl.ds(wid * bw, bw)],
            lambda i: i * 8,
            1,
            True,
        )
        for j in range(3):
            section(
                8,
                lambda i, j=j: o1_hbm.at[j, i, pl.ds(nb0, nbl)],
                lambda i, j=j: _F + i * 8 * 3 + j,
                3,
                False,
            )
        for j in range(5):
            section(
                8,
                lambda i, j=j: o2_hbm.at[j, i, pl.ds(nb0, nbl)],
                lambda i, j=j: _F * 4 + i * 8 * 5 + j,
                5,
                False,
            )

    return k(z, lut)


def kernel(Z, leq0, leq1, leq2):
    N = Z.shape[0]
    z = Z.astype(jnp.int32)
    lut = _lut_pack(leq0, leq1, leq2)
    o0, o1, o2 = _gather3(z, lut)
    out0 = jnp.transpose(o0, (2, 0, 1))
    out1 = jnp.transpose(o1, (2, 4, 1, 3, 0)).reshape(N, _F, 3)
    out2 = jnp.transpose(o2, (2, 4, 1, 3, 0)).reshape(N, _F, 5)
    return (
        out0,
        out1,
        out2,
        jnp.zeros((N, _F, 7), jnp.float32),
        jnp.zeros((N, _F, 9), jnp.float32),
        jnp.zeros((N, _F, 11), jnp.float32),
    )


# speedup vs baseline: 18.9696x; 3.8923x over previous
import functools

import jax
import jax.numpy as jnp
from jax import lax
from jax.experimental import pallas as pl
from jax.experimental.pallas import tpu as pltpu
from jax.experimental.pallas import tpu_sc as plsc

_F = 64
_NSPECIES = 14
_DIMS = (1, 3, 5)
_W = 640
_NROW = 16


def _lut_pack(leq0, leq1, leq2):
    lut = jnp.concatenate(
        [leq.reshape(_NSPECIES, _F * k) for leq, k in
         zip((leq0, leq1, leq2), _DIMS)], axis=1)
    lut = jnp.pad(lut.T, ((0, _W - lut.shape[1]), (0, _NROW - _NSPECIES)))
    return lut.reshape(-1)


def _gather3(z, lut):
    n = z.shape[0]
    info = plsc.get_sparse_core_info()
    nc, ns = info.num_cores, info.num_subcores
    nw = nc * ns
    bw = n // nw
    nbl = bw // 128
    nblocks = n // 128

    @functools.partial(
        pl.kernel,
        mesh=plsc.VectorSubcoreMesh(core_axis_name="c", subcore_axis_name="s"),
        compiler_params=pltpu.CompilerParams(
            use_tc_tiling_on_sc=False, needs_layout_passes=False),
        out_type=[
            jax.ShapeDtypeStruct((_F, 1, n), jnp.float32),
            jax.ShapeDtypeStruct((3, _F // 8, nblocks, 8, 128), jnp.float32),
            jax.ShapeDtypeStruct((5, _F // 8, nblocks, 8, 128), jnp.float32),
        ],
        scratch_types=[
            pltpu.VMEM((bw,), jnp.int32),
            pltpu.VMEM((_NROW * _W,), jnp.float32),
            pltpu.VMEM((2, 8, 8, 128), jnp.float32),
            pltpu.VMEM((2, 8, 1, 1024), jnp.float32),
            pltpu.SemaphoreType.DMA,
            pltpu.SemaphoreType.DMA,
        ],
    )
    def k(z_hbm, lut_hbm, o0_hbm, o1_hbm, o2_hbm, zv, lutv, stg, stg0,
          sem0, sem1):
        wid = lax.axis_index("s") * nc + lax.axis_index("c")
        nb0 = wid * nbl
        pltpu.sync_copy(z_hbm.at[pl.ds(wid * bw, bw)], zv)
        pltpu.sync_copy(lut_hbm, lutv)
        sems = (sem0, sem1)

        def compute(buf, col0, cstride, f_major):
            def blk(nb, carry):
                zvecs = [zv[pl.ds(nb * 128 + l * 16, 16)] for l in range(8)]
                for l in range(8):
                    vals = []
                    for fi in range(8):
                        sub = lutv.at[pl.ds((col0 + fi * cstride) * _NROW,
                                            _NROW)]
                        vals.append(plsc.load_gather(sub, [zvecs[l]]))
                    for fi, v in enumerate(vals):
                        if f_major:
                            stg0[buf, fi, 0, pl.ds(nb * 128 + l * 16, 16)] = v
                        else:
                            stg[buf, nb, fi, pl.ds(l * 16, 16)] = v
                return carry

            lax.fori_loop(0, nbl, blk, 0)

        def section(nloop, out_dst, col_of, cstride, f_major):
            buf = stg0 if f_major else stg

            def body2(g, carry):
                for par in range(2):
                    i = 2 * g + par

                    @pl.when(i >= 2)
                    def _():
                        pltpu.make_async_copy(
                            buf.at[par], out_dst(i - 2), sems[par]).wait()

                    compute(par, col_of(i), cstride, f_major)
                    pltpu.async_copy(buf.at[par], out_dst(i), sems[par])
                return carry

            lax.fori_loop(0, nloop // 2, body2, 0)
            for par in range(2):
                i = nloop - 2 + par
                pltpu.make_async_copy(buf.at[par], out_dst(i), sems[par]).wait()

        section(
            8,
            lambda i: o0_hbm.at[pl.ds(i * 8, 8), pl.ds(0, 1),
                                pl.ds(wid * bw, bw)],
            lambda i: i * 8,
            1,
            True,
        )
        for j in range(3):
            section(
                8,
                lambda i, j=j: o1_hbm.at[j, i, pl.ds(nb0, nbl)],
                lambda i, j=j: _F + i * 8 * 3 + j,
                3,
                False,
            )
        for j in range(5):
            section(
                8,
                lambda i, j=j: o2_hbm.at[j, i, pl.ds(nb0, nbl)],
                lambda i, j=j: _F * 4 + i * 8 * 5 + j,
                5,
                False,
            )

    return k(z, lut)


def kernel(Z, leq0, leq1, leq2):
    N = Z.shape[0]
    z = Z.astype(jnp.int32)
    lut = _lut_pack(leq0, leq1, leq2)
    o0, o1, o2 = _gather3(z, lut)
    out0 = jnp.transpose(o0, (2, 0, 1))
    out1 = jnp.transpose(o1, (2, 4, 1, 3, 0)).reshape(N, _F, 3)
    out2 = jnp.transpose(o2, (2, 4, 1, 3, 0)).reshape(N, _F, 5)
    return (
        out0,
        out1,
        out2,
        jnp.zeros((N, _F, 7), jnp.float32),
        jnp.zeros((N, _F, 9), jnp.float32),
        jnp.zeros((N, _F, 11), jnp.float32),
    )
